# baseline (speedup 1.0000x reference)
import jax
import jax.numpy as jnp
from jax.experimental import pallas as pl

SPAN = 128
STRIDE = 4
LQ = 2048
HEADS = 12
DQK = 64
DV = 64
DIM = 768
KV_USED = (LQ - 1) // STRIDE + 1
SCALE = 1.0 / (DQK ** 0.5)

QT = 512
WIN = 256
NT = LQ // QT


def _dot_t(a, b):
    return jax.lax.dot_general(a, b, (((1,), (1,)), ((), ())),
                               preferred_element_type=jnp.float32)


def _band_bias(rows, cols, shift):
    r = jax.lax.broadcasted_iota(jnp.int32, (rows, cols), 0)
    c4 = STRIDE * jax.lax.broadcasted_iota(jnp.int32, (rows, cols), 1)
    d = r + shift - c4
    valid = (d >= 0) & (d < SPAN)
    return jnp.where(valid, 0.0, -jnp.inf).astype(jnp.float32)


def _softmax_av(s, vwin):
    m = jnp.max(s, axis=1, keepdims=True)
    e = jnp.exp(s - m)
    denom = jnp.sum(e, axis=1, keepdims=True)
    av = jax.lax.dot_general(e, vwin, (((1,), (0,)), ((), ())),
                             preferred_element_type=jnp.float32)
    return av / denom


def _mha_kernel(q_ref, k_ref, v_ref, wq_ref, wk_ref, wv_ref, wout_ref, out_ref):
    Qf = _dot_t(q_ref[...], wq_ref[...])
    Kf = _dot_t(k_ref[...], wk_ref[...])
    Vf = _dot_t(v_ref[...], wv_ref[...])
    bias0 = _band_bias(QT, WIN // 2, 0)
    bias = _band_bias(QT, WIN, QT)
    row_tiles = []
    for t in range(NT):
        qt = Qf[t * QT:(t + 1) * QT, :]
        lo = 0 if t == 0 else WIN // 2 * (t - 1)
        w = WIN // 2 if t == 0 else WIN
        b = bias0 if t == 0 else bias
        ohs = []
        for h in range(HEADS):
            qh = qt[:, h * DQK:(h + 1) * DQK]
            kh = Kf[lo:lo + w, h * DQK:(h + 1) * DQK]
            vh = Vf[lo:lo + w, h * DV:(h + 1) * DV]
            s = _dot_t(qh, kh) * SCALE + b
            ohs.append(_softmax_av(s, vh))
        row_tiles.append(jnp.concatenate(ohs, axis=1))
    qkv = jnp.concatenate(row_tiles, axis=0).astype(jnp.bfloat16)
    out_ref[...] = _dot_t(qkv, wout_ref[...])


def kernel(q, k, v, Wq, Wk, Wv, Wout):
    batch = q.shape[0]
    bf16 = jnp.bfloat16
    q2 = q.reshape(batch * LQ, DIM).astype(bf16)
    k2 = k.reshape(-1, DIM).astype(bf16)
    v2 = v.reshape(-1, DIM).astype(bf16)
    out = pl.pallas_call(
        _mha_kernel,
        grid=(1,),
        in_specs=[
            pl.BlockSpec((LQ, DIM), lambda i: (0, 0)),
            pl.BlockSpec((KV_USED, DIM), lambda i: (0, 0)),
            pl.BlockSpec((KV_USED, DIM), lambda i: (0, 0)),
            pl.BlockSpec((HEADS * DQK, DIM), lambda i: (0, 0)),
            pl.BlockSpec((HEADS * DQK, DIM), lambda i: (0, 0)),
            pl.BlockSpec((HEADS * DV, DIM), lambda i: (0, 0)),
            pl.BlockSpec((DIM, HEADS * DV), lambda i: (0, 0)),
        ],
        out_specs=pl.BlockSpec((LQ, DIM), lambda i: (0, 0)),
        out_shape=jax.ShapeDtypeStruct((LQ, DIM), jnp.float32),
    )(q2, k2, v2, Wq.astype(bf16), Wk.astype(bf16), Wv.astype(bf16),
      Wout.astype(bf16))
    return out.reshape(batch, LQ, DIM)


# speedup vs baseline: 1.0158x; 1.0158x over previous
import jax
import jax.numpy as jnp
from jax.experimental import pallas as pl

SPAN = 128
STRIDE = 4
LQ = 2048
HEADS = 12
DQK = 64
DV = 64
DIM = 768
KV_USED = (LQ - 1) // STRIDE + 1
SCALE = 1.0 / (DQK ** 0.5)

QT = 512
WIN = 256
NT = LQ // QT


def _dot_t(a, b):
    return jax.lax.dot_general(a, b, (((1,), (1,)), ((), ())),
                               preferred_element_type=jnp.float32)


def _band_bias(rows, cols, shift):
    r = jax.lax.broadcasted_iota(jnp.int32, (rows, cols), 0)
    c4 = STRIDE * jax.lax.broadcasted_iota(jnp.int32, (rows, cols), 1)
    d = r + shift - c4
    valid = (d >= 0) & (d < SPAN)
    return jnp.where(valid, 0.0, -jnp.inf).astype(jnp.float32)


def _softmax_av(s, vwin):
    m = jnp.max(s, axis=1, keepdims=True)
    e = jnp.exp(s - m)
    denom = jnp.sum(e, axis=1, keepdims=True)
    av = jax.lax.dot_general(e, vwin, (((1,), (0,)), ((), ())),
                             preferred_element_type=jnp.float32)
    return av / denom


def _mha_kernel(q_ref, k_ref, v_ref, wq_ref, wk_ref, wv_ref, wout_ref, out_ref):
    Qf = _dot_t(q_ref[...], wq_ref[...])
    Kf = _dot_t(k_ref[...], wk_ref[...])
    Vf = _dot_t(v_ref[...], wv_ref[...])
    bias = _band_bias(LQ, KV_USED, 0)
    ohs = []
    for h in range(HEADS):
        qh = Qf[:, h * DQK:(h + 1) * DQK]
        kh = Kf[:, h * DQK:(h + 1) * DQK]
        vh = Vf[:, h * DV:(h + 1) * DV]
        s = _dot_t(qh, kh) * SCALE + bias
        ohs.append(_softmax_av(s, vh))
    qkv = jnp.concatenate(ohs, axis=1).astype(jnp.bfloat16)
    out_ref[...] = _dot_t(qkv, wout_ref[...])


def kernel(q, k, v, Wq, Wk, Wv, Wout):
    batch = q.shape[0]
    bf16 = jnp.bfloat16
    q2 = q.reshape(batch * LQ, DIM).astype(bf16)
    k2 = k.reshape(-1, DIM).astype(bf16)
    v2 = v.reshape(-1, DIM).astype(bf16)
    out = pl.pallas_call(
        _mha_kernel,
        grid=(1,),
        in_specs=[
            pl.BlockSpec((LQ, DIM), lambda i: (0, 0)),
            pl.BlockSpec((KV_USED, DIM), lambda i: (0, 0)),
            pl.BlockSpec((KV_USED, DIM), lambda i: (0, 0)),
            pl.BlockSpec((HEADS * DQK, DIM), lambda i: (0, 0)),
            pl.BlockSpec((HEADS * DQK, DIM), lambda i: (0, 0)),
            pl.BlockSpec((HEADS * DV, DIM), lambda i: (0, 0)),
            pl.BlockSpec((DIM, HEADS * DV), lambda i: (0, 0)),
        ],
        out_specs=pl.BlockSpec((LQ, DIM), lambda i: (0, 0)),
        out_shape=jax.ShapeDtypeStruct((LQ, DIM), jnp.float32),
    )(q2, k2, v2, Wq.astype(bf16), Wk.astype(bf16), Wv.astype(bf16),
      Wout.astype(bf16))
    return out.reshape(batch, LQ, DIM)


# speedup vs baseline: 1.3643x; 1.3430x over previous
import jax
import jax.numpy as jnp
from jax.experimental import pallas as pl

SPAN = 128
STRIDE = 4
LQ = 2048
HEADS = 12
DQK = 64
DV = 64
DIM = 768
KV_USED = (LQ - 1) // STRIDE + 1
SCALE = 1.0 / (DQK ** 0.5)

QT = 512
WIN = 256
NT = LQ // QT


def _dot_t(a, b):
    return jax.lax.dot_general(a, b, (((1,), (1,)), ((), ())),
                               preferred_element_type=jnp.float32)


def _band_bias(rows, cols, shift):
    r = jax.lax.broadcasted_iota(jnp.int32, (rows, cols), 0)
    c4 = STRIDE * jax.lax.broadcasted_iota(jnp.int32, (rows, cols), 1)
    d = r + shift - c4
    valid = (d >= 0) & (d < SPAN)
    return jnp.where(valid, 0.0, -jnp.inf).astype(jnp.float32)


def _softmax_av(s, vwin):
    m = jnp.max(s, axis=1, keepdims=True)
    e = jnp.exp(s - m)
    denom = jnp.sum(e, axis=1, keepdims=True)
    av = jax.lax.dot_general(e, vwin, (((1,), (0,)), ((), ())),
                             preferred_element_type=jnp.float32)
    return av / denom


def _mha_kernel(q_ref, k_ref, v_ref, wq_ref, wk_ref, wv_ref, wout_ref, out_ref):
    Qf = _dot_t(q_ref[...], wq_ref[...])
    Kf = _dot_t(k_ref[...], wk_ref[...])
    Vf = _dot_t(v_ref[...], wv_ref[...])
    bias0 = _band_bias(QT, WIN // 2, 0)
    bias = _band_bias(QT, WIN, QT)
    row_tiles = []
    for t in range(NT):
        qt = Qf[t * QT:(t + 1) * QT, :]
        lo = 0 if t == 0 else WIN // 2 * (t - 1)
        w = WIN // 2 if t == 0 else WIN
        b = bias0 if t == 0 else bias
        ohs = []
        for h in range(HEADS):
            qh = qt[:, h * DQK:(h + 1) * DQK]
            kh = Kf[lo:lo + w, h * DQK:(h + 1) * DQK]
            vh = Vf[lo:lo + w, h * DV:(h + 1) * DV]
            s = _dot_t(qh, kh) * SCALE + b
            ohs.append(_softmax_av(s, vh))
        row_tiles.append(jnp.concatenate(ohs, axis=1))
    qkv = jnp.concatenate(row_tiles, axis=0)
    out_ref[...] = _dot_t(qkv, wout_ref[...])


def kernel(q, k, v, Wq, Wk, Wv, Wout):
    batch = q.shape[0]
    q2 = q.reshape(batch * LQ, DIM)
    k2 = k.reshape(-1, DIM)
    v2 = v.reshape(-1, DIM)
    out = pl.pallas_call(
        _mha_kernel,
        grid=(1,),
        in_specs=[
            pl.BlockSpec((LQ, DIM), lambda i: (0, 0)),
            pl.BlockSpec((KV_USED, DIM), lambda i: (0, 0)),
            pl.BlockSpec((KV_USED, DIM), lambda i: (0, 0)),
            pl.BlockSpec((HEADS * DQK, DIM), lambda i: (0, 0)),
            pl.BlockSpec((HEADS * DQK, DIM), lambda i: (0, 0)),
            pl.BlockSpec((HEADS * DV, DIM), lambda i: (0, 0)),
            pl.BlockSpec((DIM, HEADS * DV), lambda i: (0, 0)),
        ],
        out_specs=pl.BlockSpec((LQ, DIM), lambda i: (0, 0)),
        out_shape=jax.ShapeDtypeStruct((LQ, DIM), jnp.float32),
    )(q2, k2, v2, Wq, Wk, Wv, Wout)
    return out.reshape(batch, LQ, DIM)


# speedup vs baseline: 1.4517x; 1.0641x over previous
import jax
import jax.numpy as jnp
from jax.experimental import pallas as pl

SPAN = 128
STRIDE = 4
LQ = 2048
HEADS = 12
DQK = 64
DV = 64
DIM = 768
KV_USED = (LQ - 1) // STRIDE + 1
SCALE = 1.0 / (DQK ** 0.5)

QT = 512
WIN = 256
NT = LQ // QT


def _dot_t(a, b):
    return jax.lax.dot_general(a, b, (((1,), (1,)), ((), ())),
                               preferred_element_type=jnp.float32)


def _band_bias(rows, cols, shift):
    r = jax.lax.broadcasted_iota(jnp.int32, (rows, cols), 0)
    c4 = STRIDE * jax.lax.broadcasted_iota(jnp.int32, (rows, cols), 1)
    d = r + shift - c4
    valid = (d >= 0) & (d < SPAN)
    return jnp.where(valid, 0.0, -jnp.inf).astype(jnp.float32)


def _softmax_av(s, vwin):
    m = jnp.max(s, axis=1, keepdims=True)
    e = jnp.exp(s - m)
    p = e / jnp.sum(e, axis=1, keepdims=True)
    return jax.lax.dot_general(p, vwin, (((1,), (0,)), ((), ())),
                               preferred_element_type=jnp.float32)


def _mha_kernel(q_ref, k_ref, v_ref, wq_ref, wk_ref, wv_ref, wout_ref, out_ref):
    Qf = _dot_t(q_ref[...], wq_ref[...])
    Kf = _dot_t(k_ref[...], wk_ref[...])
    Vf = _dot_t(v_ref[...], wv_ref[...])
    bias = _band_bias(LQ, KV_USED, 0)
    ohs = []
    for h in range(HEADS):
        qh = Qf[:, h * DQK:(h + 1) * DQK]
        kh = Kf[:, h * DQK:(h + 1) * DQK]
        vh = Vf[:, h * DV:(h + 1) * DV]
        s = _dot_t(qh, kh) * SCALE + bias
        ohs.append(_softmax_av(s, vh))
    qkv = jnp.concatenate(ohs, axis=1)
    out_ref[...] = _dot_t(qkv, wout_ref[...])


def kernel(q, k, v, Wq, Wk, Wv, Wout):
    batch = q.shape[0]
    q2 = q.reshape(batch * LQ, DIM)
    k2 = k.reshape(-1, DIM)
    v2 = v.reshape(-1, DIM)
    out = pl.pallas_call(
        _mha_kernel,
        grid=(1,),
        in_specs=[
            pl.BlockSpec((LQ, DIM), lambda i: (0, 0)),
            pl.BlockSpec((KV_USED, DIM), lambda i: (0, 0)),
            pl.BlockSpec((KV_USED, DIM), lambda i: (0, 0)),
            pl.BlockSpec((HEADS * DQK, DIM), lambda i: (0, 0)),
            pl.BlockSpec((HEADS * DQK, DIM), lambda i: (0, 0)),
            pl.BlockSpec((HEADS * DV, DIM), lambda i: (0, 0)),
            pl.BlockSpec((DIM, HEADS * DV), lambda i: (0, 0)),
        ],
        out_specs=pl.BlockSpec((LQ, DIM), lambda i: (0, 0)),
        out_shape=jax.ShapeDtypeStruct((LQ, DIM), jnp.float32),
    )(q2, k2, v2, Wq, Wk, Wv, Wout)
    return out.reshape(batch, LQ, DIM)
